# baseline (speedup 1.0000x reference)
import jax
import jax.numpy as jnp
from jax import lax
from jax.experimental import pallas as pl
from jax.experimental.pallas import tpu as pltpu
from jax.experimental.pallas import tpu_sc as plsc

N = 10000
E = 320000
HID = 32
NG = 64
NC, NS, L = 2, 16, 16
NW = NC * NS
EPW = E // NW
K = 128
NCH = E // K
CPW = 78
NXT = NCH - CPW * NW
NBUF = 6
WB = 624
WBT = N - NS * WB

_MESH = dict(core_axis_name="c", subcore_axis_name="s")


def _deg_body(dst_hbm, deg_out, idx_v, deg_v):
    wid = lax.axis_index("s") * NC + lax.axis_index("c")
    zero16 = jnp.zeros((L,), jnp.float32)

    def zero_step(i, carry):
        deg_v[0, pl.ds(i * L, L)] = zero16
        return carry

    lax.fori_loop(0, N // L, zero_step, 0)
    pltpu.sync_copy(dst_hbm.at[pl.ds(wid * EPW, EPW)], idx_v)
    ones = jnp.ones((L,), jnp.float32)
    zidx = jnp.zeros((L,), jnp.int32)

    def step(i, carry):
        idx = idx_v[pl.ds(i * L, L)]
        plsc.addupdate_scatter(deg_v, [zidx, idx], ones)
        return carry

    lax.fori_loop(0, EPW // L, step, 0)
    pltpu.sync_copy(deg_v, deg_out.at[wid])


_deg_call = pl.kernel(
    _deg_body,
    out_type=jax.ShapeDtypeStruct((NW, 1, N), jnp.float32),
    mesh=plsc.VectorSubcoreMesh(**_MESH),
    compiler_params=pltpu.CompilerParams(needs_layout_passes=False),
    scratch_types=[
        pltpu.VMEM((EPW,), jnp.int32),
        pltpu.VMEM((1, N), jnp.float32),
    ],
)


def _agg_body(srcp_hbm, dstp_hbm, xs_hbm, agg_out,
              acc_sh, xs_sh, zbuf, sidx_all, didx_all, sidx_x, didx_x,
              rows_all, gsem, ssem):
    c = lax.axis_index("c")
    s = lax.axis_index("s")
    wid = s * NC + c
    zero16 = jnp.zeros((L,), jnp.float32)

    def zero_step(i, carry):
        zbuf[i // 2, pl.ds((i % 2) * L, L)] = zero16
        return carry

    lax.fori_loop(0, WB * HID // L, zero_step, 0)
    pltpu.sync_copy(zbuf, acc_sh.at[pl.ds(s * WB, WB)])
    pltpu.sync_copy(xs_hbm.at[pl.ds(s * WB, WB)], xs_sh.at[pl.ds(s * WB, WB)])

    @pl.when(s == 0)
    def _zero_tail():
        pltpu.sync_copy(zbuf.at[pl.ds(0, WBT)],
                        acc_sh.at[pl.ds(NS * WB, WBT)])
        pltpu.sync_copy(xs_hbm.at[pl.ds(NS * WB, WBT)],
                        xs_sh.at[pl.ds(NS * WB, WBT)])

    pltpu.sync_copy(srcp_hbm.at[pl.ds(wid * CPW, CPW)], sidx_all)
    pltpu.sync_copy(dstp_hbm.at[pl.ds(wid * CPW, CPW)], didx_all)

    @pl.when(wid < NXT)
    def _extra_idx():
        pltpu.sync_copy(srcp_hbm.at[NW * CPW + wid], sidx_x)
        pltpu.sync_copy(dstp_hbm.at[NW * CPW + wid], didx_x)

    plsc.subcore_barrier()

    def _gather(g, b):
        return pltpu.make_async_copy(xs_sh.at[sidx_all.at[g]],
                                     rows_all.at[b], gsem.at[b])

    def _scatter(g, b):
        return pltpu.make_async_copy(rows_all.at[b], acc_sh.at[didx_all.at[g]],
                                     ssem.at[b])

    for b in range(NBUF):
        _gather(b, b).start()

    def group(i, carry):
        g0 = NBUF * i
        for b in range(NBUF):
            _gather(g0 + b, b).wait()
            _scatter(g0 + b, b).start(add=True)
        for b in range(NBUF):
            _scatter(g0 + b, b).wait()
            _gather(g0 + NBUF + b, b).start()
        return carry

    lax.fori_loop(0, CPW // NBUF - 1, group, 0)
    gl = CPW - NBUF
    for b in range(NBUF):
        _gather(gl + b, b).wait()
        _scatter(gl + b, b).start(add=True)
    for b in range(NBUF):
        _scatter(gl + b, b).wait()

    @pl.when(wid < NXT)
    def _extra_chunk():
        pltpu.make_async_copy(xs_sh.at[sidx_x], rows_all.at[0],
                              gsem.at[0]).start()
        pltpu.make_async_copy(xs_sh.at[sidx_x], rows_all.at[0],
                              gsem.at[0]).wait()
        pltpu.make_async_copy(rows_all.at[0], acc_sh.at[didx_x],
                              ssem.at[0]).start(add=True)
        pltpu.make_async_copy(rows_all.at[0], acc_sh.at[didx_x],
                              ssem.at[0]).wait()

    plsc.subcore_barrier()
    pltpu.sync_copy(acc_sh.at[pl.ds(s * WB, WB)],
                    agg_out.at[c, pl.ds(s * WB, WB)])

    @pl.when(s == 0)
    def _wb_tail():
        pltpu.sync_copy(acc_sh.at[pl.ds(NS * WB, WBT)],
                        agg_out.at[c, pl.ds(NS * WB, WBT)])


_agg_call = pl.kernel(
    _agg_body,
    out_type=jax.ShapeDtypeStruct((NC, N, HID), jnp.float32),
    mesh=plsc.VectorSubcoreMesh(**_MESH),
    compiler_params=pltpu.CompilerParams(needs_layout_passes=False,
                                         use_tc_tiling_on_sc=False),
    scratch_types=[
        pltpu.VMEM_SHARED((N, HID), jnp.float32),
        pltpu.VMEM_SHARED((N, HID), jnp.float32),
        pltpu.VMEM((WB, HID), jnp.float32),
        pltpu.VMEM((CPW, K), jnp.int32),
        pltpu.VMEM((CPW, K), jnp.int32),
        pltpu.VMEM((K,), jnp.int32),
        pltpu.VMEM((K,), jnp.int32),
        pltpu.VMEM((NBUF, K, HID), jnp.float32),
        pltpu.SemaphoreType.DMA((NBUF,)),
        pltpu.SemaphoreType.DMA((NBUF,)),
    ],
)


WR = N * HID // 128


def _widen_bias(b):
    return jnp.concatenate([b, b, b, b], axis=1)


def _block_diag(w):
    row = jnp.concatenate([w, w, w, w], axis=1)
    blk = jnp.concatenate([row, row, row, row], axis=0)
    ri = lax.broadcasted_iota(jnp.int32, (128, 128), 0) // HID
    ci = lax.broadcasted_iota(jnp.int32, (128, 128), 1) // HID
    return jnp.where(ri == ci, blk, 0.0)


def _prep_body(degp_ref, x_ref, w1_ref, dinvw_ref, xs1_ref):
    deg = jnp.sum(degp_ref[...], axis=0) + 1.0
    dinv = lax.rsqrt(deg).reshape(N, 1)
    xw = jnp.dot(x_ref[...], w1_ref[...], preferred_element_type=jnp.float32)
    xs = xw * dinv
    dinvw_ref[...] = jnp.concatenate(
        [jnp.broadcast_to(dinv[a * WR:(a + 1) * WR], (WR, HID))
         for a in range(4)], axis=1)
    xs1_ref[...] = jnp.concatenate(
        [xs[a * WR:(a + 1) * WR] for a in range(4)], axis=1)


def _prep_call(degp, x, w1):
    return pl.pallas_call(
        _prep_body,
        out_shape=[
            jax.ShapeDtypeStruct((WR, 128), jnp.float32),
            jax.ShapeDtypeStruct((WR, 128), jnp.float32),
        ],
    )(degp, x, w1)


def _layer_body(agg_ref, xs_ref, dw_ref, b_ref, w_ref, out_ref):
    dw = dw_ref[...]
    h = (agg_ref[0] + agg_ref[1] + xs_ref[...]) * dw + _widen_bias(b_ref[...])
    h = jnp.maximum(h, 0.0)
    out_ref[...] = jnp.dot(h, _block_diag(w_ref[...]),
                           preferred_element_type=jnp.float32) * dw


def _layer_call(agg, xs, dinvw, b, w):
    return pl.pallas_call(
        _layer_body,
        out_shape=jax.ShapeDtypeStruct((WR, 128), jnp.float32),
    )(agg, xs, dinvw, b, w)


def _final_body(agg_ref, xs_ref, dw_ref, b_ref, b4_ref, wl_ref, bl_ref,
                out_ref):
    h = (agg_ref[0] + agg_ref[1] + xs_ref[...]) * dw_ref[...] \
        + _widen_bias(b_ref[...])
    h = jnp.maximum(h, 0.0)
    gid = lax.broadcasted_iota(jnp.int32, (NG, 1), 0)
    ones = jnp.ones((WR, 1), jnp.float32)
    sums = jnp.zeros((NG, HID), jnp.float32)
    counts = jnp.zeros((NG, 1), jnp.float32)
    for a in range(4):
        oht = (gid == b4_ref[a:a + 1, :]).astype(jnp.float32)
        ha = h[:, HID * a:HID * (a + 1)]
        sums = sums + jnp.dot(oht, ha, preferred_element_type=jnp.float32)
        counts = counts + jnp.dot(oht, ones,
                                  preferred_element_type=jnp.float32)
    pooled = sums / jnp.maximum(counts, 1.0)
    out_ref[...] = jnp.dot(pooled, wl_ref[...],
                           preferred_element_type=jnp.float32) + bl_ref[...]


def _final_call(agg, xs, dinvw, b, batch4t, wl, bl):
    return pl.pallas_call(
        _final_body,
        out_shape=jax.ShapeDtypeStruct((NG, 1), jnp.float32),
    )(agg, xs, dinvw, b, batch4t, wl, bl)


def kernel(x, edge_index, batch, W1, b1, W2, b2, Wl, bl):
    src = edge_index[0].astype(jnp.int32)
    dst = edge_index[1].astype(jnp.int32)
    batch4t = batch.astype(jnp.int32).reshape(4, WR)
    srcp = ((src % WR) * 4 + src // WR).reshape(NCH, K)
    dstp = ((dst % WR) * 4 + dst // WR).reshape(NCH, K)

    degp = _deg_call(dst)
    dinvw, xs1w = _prep_call(degp, x, W1)
    agg1 = _agg_call(srcp, dstp, xs1w.reshape(N, HID))
    xs2w = _layer_call(agg1.reshape(NC, WR, 128), xs1w, dinvw,
                       b1.reshape(1, HID), W2)
    agg2 = _agg_call(srcp, dstp, xs2w.reshape(N, HID))
    return _final_call(agg2.reshape(NC, WR, 128), xs2w, dinvw,
                       b2.reshape(1, HID), batch4t, Wl, bl.reshape(1, 1))


# speedup vs baseline: 1.1766x; 1.1766x over previous
import jax
import jax.numpy as jnp
from jax import lax
from jax.experimental import pallas as pl
from jax.experimental.pallas import tpu as pltpu
from jax.experimental.pallas import tpu_sc as plsc

N = 10000
E = 320000
HID = 32
NG = 64
NC, NS, L = 2, 16, 16
NW = NC * NS
EPW = E // NW
K = 128
NCH = E // K
CPW = 78
NXT = NCH - CPW * NW
NBUF = 13
WB = 624
WBT = N - NS * WB

_MESH = dict(core_axis_name="c", subcore_axis_name="s")


def _deg_body(dst_hbm, deg_out, idx_v, deg_v):
    wid = lax.axis_index("s") * NC + lax.axis_index("c")
    zero16 = jnp.zeros((L,), jnp.float32)

    def zero_step(i, carry):
        deg_v[0, pl.ds(i * L, L)] = zero16
        return carry

    lax.fori_loop(0, N // L, zero_step, 0)
    pltpu.sync_copy(dst_hbm.at[pl.ds(wid * EPW, EPW)], idx_v)
    ones = jnp.ones((L,), jnp.float32)
    zidx = jnp.zeros((L,), jnp.int32)

    def step(i, carry):
        idx = idx_v[pl.ds(i * L, L)]
        plsc.addupdate_scatter(deg_v, [zidx, idx], ones)
        return carry

    lax.fori_loop(0, EPW // L, step, 0)
    pltpu.sync_copy(deg_v, deg_out.at[wid])


_deg_call = pl.kernel(
    _deg_body,
    out_type=jax.ShapeDtypeStruct((NW, 1, N), jnp.float32),
    mesh=plsc.VectorSubcoreMesh(**_MESH),
    compiler_params=pltpu.CompilerParams(needs_layout_passes=False),
    scratch_types=[
        pltpu.VMEM((EPW,), jnp.int32),
        pltpu.VMEM((1, N), jnp.float32),
    ],
)


def _agg_body(srcp_hbm, dstp_hbm, xs_hbm, agg_out,
              acc_sh, zbuf, sidx_all, didx_all, sidx_x, didx_x,
              rows_all, gsem, ssem):
    c = lax.axis_index("c")
    s = lax.axis_index("s")
    wid = s * NC + c
    zero16 = jnp.zeros((L,), jnp.float32)

    def zero_step(i, carry):
        zbuf[i // 2, pl.ds((i % 2) * L, L)] = zero16
        return carry

    lax.fori_loop(0, WB * HID // L, zero_step, 0)
    pltpu.sync_copy(zbuf, acc_sh.at[pl.ds(s * WB, WB)])

    @pl.when(s == 0)
    def _zero_tail():
        pltpu.sync_copy(zbuf.at[pl.ds(0, WBT)],
                        acc_sh.at[pl.ds(NS * WB, WBT)])

    pltpu.sync_copy(srcp_hbm.at[pl.ds(wid * CPW, CPW)], sidx_all)
    pltpu.sync_copy(dstp_hbm.at[pl.ds(wid * CPW, CPW)], didx_all)

    @pl.when(wid < NXT)
    def _extra_idx():
        pltpu.sync_copy(srcp_hbm.at[NW * CPW + wid], sidx_x)
        pltpu.sync_copy(dstp_hbm.at[NW * CPW + wid], didx_x)

    plsc.subcore_barrier()

    def _gather(g, b):
        return pltpu.make_async_copy(xs_hbm.at[sidx_all.at[g]],
                                     rows_all.at[b], gsem.at[b])

    def _scatter(g, b):
        return pltpu.make_async_copy(rows_all.at[b], acc_sh.at[didx_all.at[g]],
                                     ssem.at[b])

    for b in range(NBUF):
        _gather(b, b).start()

    def group(i, carry):
        g0 = NBUF * i
        for b in range(NBUF):
            _gather(g0 + b, b).wait()
            _scatter(g0 + b, b).start(add=True)
        for b in range(NBUF):
            _scatter(g0 + b, b).wait()
            _gather(g0 + NBUF + b, b).start()
        return carry

    lax.fori_loop(0, CPW // NBUF - 1, group, 0)
    gl = CPW - NBUF
    for b in range(NBUF):
        _gather(gl + b, b).wait()
        _scatter(gl + b, b).start(add=True)
    for b in range(NBUF):
        _scatter(gl + b, b).wait()

    @pl.when(wid < NXT)
    def _extra_chunk():
        pltpu.make_async_copy(xs_hbm.at[sidx_x], rows_all.at[0],
                              gsem.at[0]).start()
        pltpu.make_async_copy(xs_hbm.at[sidx_x], rows_all.at[0],
                              gsem.at[0]).wait()
        pltpu.make_async_copy(rows_all.at[0], acc_sh.at[didx_x],
                              ssem.at[0]).start(add=True)
        pltpu.make_async_copy(rows_all.at[0], acc_sh.at[didx_x],
                              ssem.at[0]).wait()

    plsc.subcore_barrier()
    pltpu.sync_copy(acc_sh.at[pl.ds(s * WB, WB)],
                    agg_out.at[c, pl.ds(s * WB, WB)])

    @pl.when(s == 0)
    def _wb_tail():
        pltpu.sync_copy(acc_sh.at[pl.ds(NS * WB, WBT)],
                        agg_out.at[c, pl.ds(NS * WB, WBT)])


_agg_call = pl.kernel(
    _agg_body,
    out_type=jax.ShapeDtypeStruct((NC, N, HID), jnp.float32),
    mesh=plsc.VectorSubcoreMesh(**_MESH),
    compiler_params=pltpu.CompilerParams(needs_layout_passes=False,
                                         use_tc_tiling_on_sc=False),
    scratch_types=[
        pltpu.VMEM_SHARED((N, HID), jnp.float32),
        pltpu.VMEM((WB, HID), jnp.float32),
        pltpu.VMEM((CPW, K), jnp.int32),
        pltpu.VMEM((CPW, K), jnp.int32),
        pltpu.VMEM((K,), jnp.int32),
        pltpu.VMEM((K,), jnp.int32),
        pltpu.VMEM((NBUF, K, HID), jnp.float32),
        pltpu.SemaphoreType.DMA((NBUF,)),
        pltpu.SemaphoreType.DMA((NBUF,)),
    ],
)


WR = N * HID // 128


def _widen_bias(b):
    return jnp.concatenate([b, b, b, b], axis=1)


def _block_diag(w):
    row = jnp.concatenate([w, w, w, w], axis=1)
    blk = jnp.concatenate([row, row, row, row], axis=0)
    ri = lax.broadcasted_iota(jnp.int32, (128, 128), 0) // HID
    ci = lax.broadcasted_iota(jnp.int32, (128, 128), 1) // HID
    return jnp.where(ri == ci, blk, 0.0)


def _prep_body(degp_ref, x_ref, w1_ref, dinvw_ref, xs1_ref):
    deg = jnp.sum(degp_ref[...], axis=0) + 1.0
    dinv = lax.rsqrt(deg).reshape(N, 1)
    xw = jnp.dot(x_ref[...], w1_ref[...], preferred_element_type=jnp.float32)
    xs = xw * dinv
    dinvw_ref[...] = jnp.concatenate(
        [jnp.broadcast_to(dinv[a * WR:(a + 1) * WR], (WR, HID))
         for a in range(4)], axis=1)
    xs1_ref[...] = jnp.concatenate(
        [xs[a * WR:(a + 1) * WR] for a in range(4)], axis=1)


def _prep_call(degp, x, w1):
    return pl.pallas_call(
        _prep_body,
        out_shape=[
            jax.ShapeDtypeStruct((WR, 128), jnp.float32),
            jax.ShapeDtypeStruct((WR, 128), jnp.float32),
        ],
    )(degp, x, w1)


def _layer_body(agg_ref, xs_ref, dw_ref, b_ref, w_ref, out_ref):
    dw = dw_ref[...]
    h = (agg_ref[0] + agg_ref[1] + xs_ref[...]) * dw + _widen_bias(b_ref[...])
    h = jnp.maximum(h, 0.0)
    out_ref[...] = jnp.dot(h, _block_diag(w_ref[...]),
                           preferred_element_type=jnp.float32) * dw


def _layer_call(agg, xs, dinvw, b, w):
    return pl.pallas_call(
        _layer_body,
        out_shape=jax.ShapeDtypeStruct((WR, 128), jnp.float32),
    )(agg, xs, dinvw, b, w)


def _final_body(agg_ref, xs_ref, dw_ref, b_ref, b4_ref, wl_ref, bl_ref,
                out_ref):
    h = (agg_ref[0] + agg_ref[1] + xs_ref[...]) * dw_ref[...] \
        + _widen_bias(b_ref[...])
    h = jnp.maximum(h, 0.0)
    gid = lax.broadcasted_iota(jnp.int32, (NG, 1), 0)
    ones = jnp.ones((WR, 1), jnp.float32)
    sums = jnp.zeros((NG, HID), jnp.float32)
    counts = jnp.zeros((NG, 1), jnp.float32)
    for a in range(4):
        oht = (gid == b4_ref[a:a + 1, :]).astype(jnp.float32)
        ha = h[:, HID * a:HID * (a + 1)]
        sums = sums + jnp.dot(oht, ha, preferred_element_type=jnp.float32)
        counts = counts + jnp.dot(oht, ones,
                                  preferred_element_type=jnp.float32)
    pooled = sums / jnp.maximum(counts, 1.0)
    out_ref[...] = jnp.dot(pooled, wl_ref[...],
                           preferred_element_type=jnp.float32) + bl_ref[...]


def _final_call(agg, xs, dinvw, b, batch4t, wl, bl):
    return pl.pallas_call(
        _final_body,
        out_shape=jax.ShapeDtypeStruct((NG, 1), jnp.float32),
    )(agg, xs, dinvw, b, batch4t, wl, bl)


def kernel(x, edge_index, batch, W1, b1, W2, b2, Wl, bl):
    src = edge_index[0].astype(jnp.int32)
    dst = edge_index[1].astype(jnp.int32)
    batch4t = batch.astype(jnp.int32).reshape(4, WR)
    srcp = ((src % WR) * 4 + src // WR).reshape(NCH, K)
    dstp = ((dst % WR) * 4 + dst // WR).reshape(NCH, K)

    degp = _deg_call(dst)
    dinvw, xs1w = _prep_call(degp, x, W1)
    agg1 = _agg_call(srcp, dstp, xs1w.reshape(N, HID))
    xs2w = _layer_call(agg1.reshape(NC, WR, 128), xs1w, dinvw,
                       b1.reshape(1, HID), W2)
    agg2 = _agg_call(srcp, dstp, xs2w.reshape(N, HID))
    return _final_call(agg2.reshape(NC, WR, 128), xs2w, dinvw,
                       b2.reshape(1, HID), batch4t, Wl, bl.reshape(1, 1))
